# baseline (speedup 1.0000x reference)
import functools

import jax
import jax.numpy as jnp
from jax import lax
from jax.experimental import pallas as pl
from jax.experimental.pallas import tpu as pltpu
from jax.experimental.pallas import tpu_sc as plsc

V = 100000
WIN = 5
B = 1024
L = 200
EMB = 32
NWIN = L - WIN + 1

NC, NS = 2, 16
NWORK = NC * NS
SEQ_PER_W = B // NWORK

SEQ_PAD = 224
CHUNK = 112
NCH_R = WIN * SEQ_PAD // CHUNK
NCH_W = SEQ_PAD // CHUNK
NIDX = WIN * SEQ_PAD


def _body(seq_hbm, wr_hbm, ww_hbm, out_hbm, seq_pad, idx_v, rows_v, word_v, out_v, sem):
    wid = lax.axis_index("s") * NC + lax.axis_index("c")

    def per_seq(s, carry):
        b = wid * SEQ_PER_W + s
        zeros = jnp.zeros((16,), jnp.int32)
        seq_pad[pl.ds(L - 8, 16)] = zeros
        seq_pad[pl.ds(L + 8, 16)] = zeros
        pltpu.sync_copy(seq_hbm.at[pl.ds(b * L, L)], seq_pad.at[pl.ds(0, L)])

        for i in range(WIN):
            for k in range(SEQ_PAD // 16):
                idx_v[pl.ds(i * SEQ_PAD + k * 16, 16)] = (
                    seq_pad[pl.ds(k * 16, 16)] + (i * V)
                )

        copies = []
        for c in range(NCH_R):
            copies.append(pltpu.async_copy(
                wr_hbm.at[idx_v.at[pl.ds(c * CHUNK, CHUNK)]],
                rows_v.at[pl.ds(c * CHUNK, CHUNK)], sem))
        for c in range(NCH_W):
            copies.append(pltpu.async_copy(
                ww_hbm.at[seq_pad.at[pl.ds(c * CHUNK, CHUNK)]],
                word_v.at[pl.ds(c * CHUNK, CHUNK)], sem))
        for cp in copies:
            cp.wait()

        @plsc.parallel_loop(0, NWIN, 1, unroll=4)
        def _win(p):
            w0 = word_v[p + WIN // 2, pl.ds(0, 16)]
            w1 = word_v[p + WIN // 2, pl.ds(16, 16)]
            a0 = rows_v[p, pl.ds(0, 16)] * w0
            a1 = rows_v[p, pl.ds(16, 16)] * w1
            for i in range(1, WIN):
                r = p + i * SEQ_PAD + i
                a0 = jnp.maximum(a0, rows_v[r, pl.ds(0, 16)] * w0)
                a1 = jnp.maximum(a1, rows_v[r, pl.ds(16, 16)] * w1)
            out_v[pl.ds(p * EMB, 16)] = a0
            out_v[pl.ds(p * EMB + 16, 16)] = a1
        pltpu.sync_copy(out_v, out_hbm.at[pl.ds(b * (NWIN * EMB), NWIN * EMB)])
        return carry

    lax.fori_loop(0, SEQ_PER_W, per_seq, 0)


@jax.jit
def _run(seq, W_region, W_word):
    f = pl.kernel(
        _body,
        out_type=jax.ShapeDtypeStruct((B * NWIN * EMB,), jnp.float32),
        mesh=plsc.VectorSubcoreMesh(
            core_axis_name="c", subcore_axis_name="s",
            num_cores=NC, num_subcores=NS),
        scratch_types=[
            pltpu.VMEM((SEQ_PAD,), jnp.int32),
            pltpu.VMEM((NIDX,), jnp.int32),
            pltpu.VMEM((NIDX, EMB), jnp.float32),
            pltpu.VMEM((SEQ_PAD, EMB), jnp.float32),
            pltpu.VMEM((NWIN * EMB,), jnp.float32),
            pltpu.SemaphoreType.DMA,
        ],
        compiler_params=pltpu.CompilerParams(use_tc_tiling_on_sc=False),
    )
    out = f(seq.reshape(B * L), W_region, W_word)
    return out.reshape(B, NWIN, EMB)


def kernel(seq, W_region, W_word):
    return _run(seq.astype(jnp.int32), W_region, W_word)


# speedup vs baseline: 1.5756x; 1.5756x over previous
import jax
import jax.numpy as jnp
from jax import lax
from jax.experimental import pallas as pl
from jax.experimental.pallas import tpu as pltpu
from jax.experimental.pallas import tpu_sc as plsc

V = 100000
WIN = 5
B = 1024
L = 200
EMB = 32
NWIN = L - WIN + 1
OUT_ROW = NWIN * EMB

NC, NS = 2, 16
NWORK = NC * NS
SEQ_PER_W = B // NWORK

SEQ_PAD = 224
NIDX = WIN * SEQ_PAD
SEQ_ALL = SEQ_PER_W * L


def _body(seq_hbm, wr_hbm, ww_hbm, out_hbm, seq_all,
          idx0, idx1, rows0, rows1, word0, word1, out0, out1, sem0, sem1):
    bufs = ((idx0, rows0, word0, out0, sem0),
            (idx1, rows1, word1, out1, sem1))
    wid = lax.axis_index("s") * NC + lax.axis_index("c")

    zeros = jnp.zeros((16,), jnp.int32)
    seq_all[pl.ds(SEQ_ALL, 16)] = zeros
    seq_all[pl.ds(SEQ_ALL + 16, 16)] = zeros
    pltpu.sync_copy(seq_hbm.at[pl.ds(wid * SEQ_ALL, SEQ_ALL)],
                    seq_all.at[pl.ds(0, SEQ_ALL)])

    def issue(s, buf):
        idx, rows, word, _, sem = buf
        for i in range(WIN):
            for k in range(SEQ_PAD // 16):
                idx[pl.ds(i * SEQ_PAD + k * 16, 16)] = (
                    seq_all[pl.ds(s * L + k * 16, 16)] + (i * V)
                )
        pltpu.async_copy(wr_hbm.at[idx], rows, sem)
        pltpu.async_copy(ww_hbm.at[idx.at[pl.ds(0, SEQ_PAD)]], word, sem)

    def drain(buf):
        idx, rows, word, _, sem = buf
        pltpu.make_async_copy(wr_hbm.at[idx], rows, sem).wait()
        pltpu.make_async_copy(ww_hbm.at[idx.at[pl.ds(0, SEQ_PAD)]], word, sem).wait()

    def compute(s, buf):
        _, rows, word, out, _ = buf

        @plsc.parallel_loop(0, NWIN, 1, unroll=4)
        def _win(p):
            w0 = word[p + WIN // 2, pl.ds(0, 16)]
            w1 = word[p + WIN // 2, pl.ds(16, 16)]
            a0 = rows[p, pl.ds(0, 16)] * w0
            a1 = rows[p, pl.ds(16, 16)] * w1
            for i in range(1, WIN):
                r = p + i * SEQ_PAD + i
                a0 = jnp.maximum(a0, rows[r, pl.ds(0, 16)] * w0)
                a1 = jnp.maximum(a1, rows[r, pl.ds(16, 16)] * w1)
            out[pl.ds(p * EMB, 16)] = a0
            out[pl.ds(p * EMB + 16, 16)] = a1

        pltpu.sync_copy(out, out_hbm.at[pl.ds((wid * SEQ_PER_W + s) * OUT_ROW,
                                              OUT_ROW)])

    issue(0, bufs[0])

    def outer(h, carry):
        for par in (0, 1):
            g = 2 * h + par

            @pl.when(g + 1 < SEQ_PER_W)
            def _():
                issue(g + 1, bufs[1 - par])

            drain(bufs[par])
            compute(g, bufs[par])
        return carry

    lax.fori_loop(0, SEQ_PER_W // 2, outer, 0)


@jax.jit
def _run(seq, W_region, W_word):
    f = pl.kernel(
        _body,
        out_type=jax.ShapeDtypeStruct((B * NWIN * EMB,), jnp.float32),
        mesh=plsc.VectorSubcoreMesh(
            core_axis_name="c", subcore_axis_name="s",
            num_cores=NC, num_subcores=NS),
        scratch_types=[
            pltpu.VMEM((SEQ_ALL + 32,), jnp.int32),
            pltpu.VMEM((NIDX,), jnp.int32),
            pltpu.VMEM((NIDX,), jnp.int32),
            pltpu.VMEM((NIDX, EMB), jnp.float32),
            pltpu.VMEM((NIDX, EMB), jnp.float32),
            pltpu.VMEM((SEQ_PAD, EMB), jnp.float32),
            pltpu.VMEM((SEQ_PAD, EMB), jnp.float32),
            pltpu.VMEM((NWIN * EMB,), jnp.float32),
            pltpu.VMEM((NWIN * EMB,), jnp.float32),
            pltpu.SemaphoreType.DMA,
            pltpu.SemaphoreType.DMA,
        ],
        compiler_params=pltpu.CompilerParams(use_tc_tiling_on_sc=False),
    )
    out = f(seq.reshape(B * L), W_region, W_word)
    return out.reshape(B, NWIN, EMB)


def kernel(seq, W_region, W_word):
    return _run(seq.astype(jnp.int32), W_region, W_word)
